# baseline (speedup 1.0000x reference)
import jax
import jax.numpy as jnp
from jax import lax
from jax.experimental import pallas as pl
from jax.experimental.pallas import tpu as pltpu
from jax.experimental.pallas import tpu_sc as plsc

B = 8
NN = 512
EN = 1024
K = 16
D = 768
V = 8192
MAXLEN = 2048
ACT = 1024
NR = B * NN
HB = 4
C = 64
NCH = 1
TPW = NCH * C
NBUF = 2


def _sc_body(nfid_h, efid_h, atom_h, edge_h, feat_h,
             idxa, idxe, b00, b10,
             ga0, ga1, w0, w1):
    cid = lax.axis_index("c")
    sid = lax.axis_index("s")
    wid = sid * 2 + cid
    b = wid // 8
    qq = wid % 8
    nb = b * ACT + qq * TPW
    eb = nb + NN
    irow = wid * NCH

    pltpu.sync_copy(nfid_h.at[pl.ds(irow, NCH)], idxa)
    pltpu.sync_copy(efid_h.at[pl.ds(irow, NCH)], idxe)

    bufs = [b00, b10]
    gsem = [ga0, ga1]
    wsem = [w0, w1]

    def issue(i):
        s = i % NBUF
        if i < NCH:
            return pltpu.async_copy(atom_h.at[idxa.at[i]], bufs[s], gsem[s])
        return pltpu.async_copy(edge_h.at[idxe.at[i - NCH]], bufs[s], gsem[s])

    def wrow(i):
        if i < NCH:
            return feat_h.at[pl.ds(nb + i * C, C)]
        return feat_h.at[pl.ds(eb + (i - NCH) * C, C)]

    NTOT = 2 * NCH
    gd = [None] * NBUF
    wd = [None] * NBUF
    for p in range(min(NBUF - 1, NTOT)):
        gd[p] = issue(p)
    for i in range(NTOT):
        s = i % NBUF
        nxt = i + NBUF - 1
        if nxt < NTOT:
            ns = nxt % NBUF
            if wd[ns] is not None:
                wd[ns].wait()
                wd[ns] = None
            gd[ns] = issue(nxt)
        gd[s].wait()
        wd[s] = pltpu.async_copy(bufs[s], wrow(i), wsem[s])
    for d in wd:
        if d is not None:
            d.wait()


def _tc_zero_body(out_ref):
    out_ref[...] = jnp.zeros((1, ACT, D), jnp.float32)


def _tc_body(lapn_ref, euv_ref, feat_ref,
             w_ref, ord_ref, zo_ref, out_ref):
    del zo_ref
    lapn = lapn_ref[...]
    uu = euv_ref[:, 0:1]
    vv = euv_ref[:, 8:9]
    col = lax.broadcasted_iota(jnp.int32, (NN, NN), 1)
    e0 = jnp.dot((uu == col).astype(jnp.float32), lapn,
                 preferred_element_type=jnp.float32)
    e1 = jnp.dot((vv == col).astype(jnp.float32), lapn,
                 preferred_element_type=jnp.float32)
    u = jnp.concatenate([lapn, e0], axis=0)
    v = jnp.concatenate([lapn, e1], axis=0)
    eqf = jnp.concatenate(
        [jnp.ones((NN, 1), jnp.float32),
         (uu == vv).astype(jnp.float32)], axis=0)
    wm = w_ref[...]
    dn = (((1,), (1,)), ((), ()))
    le = (lax.dot_general(u, wm[:, :K], dn,
                          preferred_element_type=jnp.float32)
          + lax.dot_general(v, wm[:, K:], dn,
                            preferred_element_type=jnp.float32))
    o0 = ord_ref[0:1, :]
    oe = o0 + eqf * (ord_ref[1:2, :] - o0)
    out_ref[0] = feat_ref[...] + le + oe


def _sc_gather(nfid, efid, atom_emb, edge_emb):
    mesh = plsc.VectorSubcoreMesh(core_axis_name="c", subcore_axis_name="s")
    return pl.kernel(
        _sc_body,
        out_type=jax.ShapeDtypeStruct((HB * ACT, D), jnp.float32),
        mesh=mesh,
        scratch_types=[
            pltpu.VMEM((NCH, C), jnp.int32),
            pltpu.VMEM((NCH, C), jnp.int32),
            pltpu.VMEM((C, D), jnp.float32),
            pltpu.VMEM((C, D), jnp.float32),
            pltpu.SemaphoreType.DMA,
            pltpu.SemaphoreType.DMA,
            pltpu.SemaphoreType.DMA,
            pltpu.SemaphoreType.DMA,
        ],
    )(nfid, efid, atom_emb, edge_emb)


def _tc_combine(half, lapf, euv, feat, lapw, ordm, prev):
    off = half * HB
    return pl.pallas_call(
        _tc_body,
        grid=(HB,),
        in_specs=[
            pl.BlockSpec((NN, K), lambda b: (b + off, 0)),
            pl.BlockSpec((NN, 16), lambda b: (b + off, 0)),
            pl.BlockSpec((ACT, D), lambda b: (b, 0)),
            pl.BlockSpec((D, 2 * K), lambda b: (0, 0)),
            pl.BlockSpec((2, D), lambda b: (0, 0)),
            pl.BlockSpec(memory_space=pl.ANY),
        ],
        out_specs=pl.BlockSpec((1, ACT, D), lambda b: (b + off, 0, 0)),
        out_shape=jax.ShapeDtypeStruct((B, MAXLEN, D), jnp.float32),
        input_output_aliases={5: 0},
    )(lapf, euv, feat, lapw, ordm, prev)


def kernel(node_data, node_num, lap_eigvec, edge_index, edge_data, edge_num,
           atom_emb, edge_emb, lap_W, order_emb):
    nfid = node_data.reshape(B * NN // C, C).astype(jnp.int32)
    efid = edge_data.reshape(B, EN)[:, :NN].reshape(B * NN // C, C).astype(jnp.int32)
    HROWS = HB * NN // C

    ei = edge_index.astype(jnp.int32)
    euv = jnp.repeat(
        ei.reshape(2, B, EN)[:, :, :NN].transpose(1, 2, 0).reshape(NR, 2),
        8, axis=1)

    lapf = lap_eigvec.astype(jnp.float32)
    lapw = lap_W.astype(jnp.float32)
    ordm = order_emb.astype(jnp.float32)

    feat_a = _sc_gather(nfid[:HROWS], efid[:HROWS], atom_emb, edge_emb)
    feat_b = _sc_gather(nfid[HROWS:], efid[HROWS:], atom_emb, edge_emb)

    zhalf = pl.pallas_call(
        _tc_zero_body,
        grid=(B,),
        out_specs=pl.BlockSpec((1, ACT, D), lambda b: (b, 1, 0)),
        out_shape=jax.ShapeDtypeStruct((B, MAXLEN, D), jnp.float32),
    )()

    out = _tc_combine(0, lapf, euv, feat_a, lapw, ordm, zhalf)
    out = _tc_combine(1, lapf, euv, feat_b, lapw, ordm, out)
    return out


# speedup vs baseline: 1.0674x; 1.0674x over previous
import jax
import jax.numpy as jnp
from jax import lax
from jax.experimental import pallas as pl
from jax.experimental.pallas import tpu as pltpu
from jax.experimental.pallas import tpu_sc as plsc

B = 8
NN = 512
EN = 1024
K = 16
D = 768
V = 8192
MAXLEN = 2048
ACT = 1024
NR = B * NN
HB = 4
C = 64
NCH = 1
TPW = NCH * C
NBUF = 2


def _make_sc_body(half):
    def _sc_body(nd_h, ed_h, atom_h, edge_h, feat_h,
                 idxa, idxe, b00, b10,
                 ga0, ga1, w0, w1):
        cid = lax.axis_index("c")
        sid = lax.axis_index("s")
        wid = sid * 2 + cid
        b = wid // 8
        qq = wid % 8
        nb = b * ACT + qq * TPW
        eb = nb + NN
        noff = (half * HB + b) * NN + qq * TPW
        eoff = (half * HB + b) * EN + qq * TPW
        pltpu.sync_copy(nd_h.at[pl.ds(noff, TPW)], idxa)
        pltpu.sync_copy(ed_h.at[pl.ds(eoff, TPW)], idxe)

        bufs = [b00, b10]
        gsem = [ga0, ga1]
        wsem = [w0, w1]

        def issue(i):
            s = i % NBUF
            if i < NCH:
                return pltpu.async_copy(atom_h.at[idxa], bufs[s], gsem[s])
            return pltpu.async_copy(edge_h.at[idxe], bufs[s], gsem[s])

        def wrow(i):
            if i < NCH:
                return feat_h.at[pl.ds(nb + i * C, C)]
            return feat_h.at[pl.ds(eb + (i - NCH) * C, C)]

        NTOT = 2 * NCH
        gd = [None] * NBUF
        wd = [None] * NBUF
        for p in range(min(NBUF - 1, NTOT)):
            gd[p] = issue(p)
        for i in range(NTOT):
            s = i % NBUF
            nxt = i + NBUF - 1
            if nxt < NTOT:
                ns = nxt % NBUF
                if wd[ns] is not None:
                    wd[ns].wait()
                    wd[ns] = None
                gd[ns] = issue(nxt)
            gd[s].wait()
            wd[s] = pltpu.async_copy(bufs[s], wrow(i), wsem[s])
        for d in wd:
            if d is not None:
                d.wait()

    return _sc_body


def _tc_zero_body(out_ref):
    out_ref[...] = jnp.zeros((1, ACT, D), jnp.float32)


def _tc_body(lapn_ref, euv_ref, feat_ref,
             w_ref, ord_ref, zo_ref, out_ref):
    del zo_ref
    lapn = lapn_ref[...]
    uu = euv_ref[:, 0:1]
    vv = euv_ref[:, 8:9]
    col = lax.broadcasted_iota(jnp.int32, (NN, NN), 1)
    e0 = jnp.dot((uu == col).astype(jnp.float32), lapn,
                 preferred_element_type=jnp.float32)
    e1 = jnp.dot((vv == col).astype(jnp.float32), lapn,
                 preferred_element_type=jnp.float32)
    u = jnp.concatenate([lapn, e0], axis=0)
    v = jnp.concatenate([lapn, e1], axis=0)
    eqf = jnp.concatenate(
        [jnp.ones((NN, 1), jnp.float32),
         (uu == vv).astype(jnp.float32)], axis=0)
    wm = w_ref[...]
    dn = (((1,), (1,)), ((), ()))
    le = (lax.dot_general(u, wm[:, :K], dn,
                          preferred_element_type=jnp.float32)
          + lax.dot_general(v, wm[:, K:], dn,
                            preferred_element_type=jnp.float32))
    o0 = ord_ref[0:1, :]
    oe = o0 + eqf * (ord_ref[1:2, :] - o0)
    out_ref[0] = feat_ref[...] + le + oe


def _sc_gather(half, nd, ed, atom_emb, edge_emb):
    mesh = plsc.VectorSubcoreMesh(core_axis_name="c", subcore_axis_name="s")
    return pl.kernel(
        _make_sc_body(half),
        out_type=jax.ShapeDtypeStruct((HB * ACT, D), jnp.float32),
        mesh=mesh,
        scratch_types=[
            pltpu.VMEM((TPW,), jnp.int32),
            pltpu.VMEM((TPW,), jnp.int32),
            pltpu.VMEM((C, D), jnp.float32),
            pltpu.VMEM((C, D), jnp.float32),
            pltpu.SemaphoreType.DMA,
            pltpu.SemaphoreType.DMA,
            pltpu.SemaphoreType.DMA,
            pltpu.SemaphoreType.DMA,
        ],
    )(nd, ed, atom_emb, edge_emb)


def _tc_combine(half, lapf, euv, feat, lapw, ordm, prev):
    off = half * HB
    return pl.pallas_call(
        _tc_body,
        grid=(HB,),
        in_specs=[
            pl.BlockSpec((NN, K), lambda b: (b + off, 0)),
            pl.BlockSpec((NN, 16), lambda b: (b + off, 0)),
            pl.BlockSpec((ACT, D), lambda b: (b, 0)),
            pl.BlockSpec((D, 2 * K), lambda b: (0, 0)),
            pl.BlockSpec((2, D), lambda b: (0, 0)),
            pl.BlockSpec(memory_space=pl.ANY),
        ],
        out_specs=pl.BlockSpec((1, ACT, D), lambda b: (b + off, 0, 0)),
        out_shape=jax.ShapeDtypeStruct((B, MAXLEN, D), jnp.float32),
        input_output_aliases={5: 0},
    )(lapf, euv, feat, lapw, ordm, prev)


def kernel(node_data, node_num, lap_eigvec, edge_index, edge_data, edge_num,
           atom_emb, edge_emb, lap_W, order_emb):
    nd = node_data.reshape(B * NN).astype(jnp.int32)
    ed = edge_data.reshape(B * EN).astype(jnp.int32)

    ei = edge_index.astype(jnp.int32)
    euv = jnp.repeat(
        ei.reshape(2, B, EN)[:, :, :NN].transpose(1, 2, 0).reshape(NR, 2),
        8, axis=1)

    lapf = lap_eigvec.astype(jnp.float32)
    lapw = lap_W.astype(jnp.float32)
    ordm = order_emb.astype(jnp.float32)

    feat_a = _sc_gather(0, nd, ed, atom_emb, edge_emb)
    feat_b = _sc_gather(1, nd, ed, atom_emb, edge_emb)

    zhalf = pl.pallas_call(
        _tc_zero_body,
        grid=(B,),
        out_specs=pl.BlockSpec((1, ACT, D), lambda b: (b, 1, 0)),
        out_shape=jax.ShapeDtypeStruct((B, MAXLEN, D), jnp.float32),
    )()

    out = _tc_combine(0, lapf, euv, feat_a, lapw, ordm, zhalf)
    out = _tc_combine(1, lapf, euv, feat_b, lapw, ordm, out)
    return out


# speedup vs baseline: 1.0676x; 1.0001x over previous
import jax
import jax.numpy as jnp
from jax import lax
from jax.experimental import pallas as pl
from jax.experimental.pallas import tpu as pltpu
from jax.experimental.pallas import tpu_sc as plsc

B = 8
NN = 512
EN = 1024
K = 16
D = 768
V = 8192
MAXLEN = 2048
ACT = 1024
NR = B * NN
HB = 4
C = 64
NCH = 1
TPW = NCH * C
NBUF = 2


def _make_sc_body(half):
    def _sc_body(nd_h, ed_h, atom_h, edge_h, feat_h,
                 idxa, idxe, b00, b10,
                 ga0, ga1, w0, w1):
        cid = lax.axis_index("c")
        sid = lax.axis_index("s")
        wid = sid * 2 + cid
        b = wid // 8
        qq = wid % 8
        nb = b * ACT + qq * TPW
        eb = nb + NN
        noff = (half * HB + b) * NN + qq * TPW
        eoff = (half * HB + b) * EN + qq * TPW
        ia = pltpu.async_copy(nd_h.at[pl.ds(noff, TPW)], idxa, w0)
        ie = pltpu.async_copy(ed_h.at[pl.ds(eoff, TPW)], idxe, w1)
        ia.wait()
        ie.wait()

        bufs = [b00, b10]
        gsem = [ga0, ga1]
        wsem = [w0, w1]

        def issue(i):
            s = i % NBUF
            if i < NCH:
                return pltpu.async_copy(atom_h.at[idxa], bufs[s], gsem[s])
            return pltpu.async_copy(edge_h.at[idxe], bufs[s], gsem[s])

        def wrow(i):
            if i < NCH:
                return feat_h.at[pl.ds(nb + i * C, C)]
            return feat_h.at[pl.ds(eb + (i - NCH) * C, C)]

        NTOT = 2 * NCH
        gd = [None] * NBUF
        wd = [None] * NBUF
        for p in range(min(NBUF - 1, NTOT)):
            gd[p] = issue(p)
        for i in range(NTOT):
            s = i % NBUF
            nxt = i + NBUF - 1
            if nxt < NTOT:
                ns = nxt % NBUF
                if wd[ns] is not None:
                    wd[ns].wait()
                    wd[ns] = None
                gd[ns] = issue(nxt)
            gd[s].wait()
            wd[s] = pltpu.async_copy(bufs[s], wrow(i), wsem[s])
        for d in wd:
            if d is not None:
                d.wait()

    return _sc_body


def _tc_zero_body(out_ref):
    out_ref[...] = jnp.zeros((1, ACT, D), jnp.float32)


def _tc_body(lapn_ref, euv_ref, feat_ref,
             w_ref, ord_ref, zo_ref, out_ref):
    del zo_ref
    lapn = lapn_ref[...]
    uu = euv_ref[:, 0:1]
    vv = euv_ref[:, 8:9]
    col = lax.broadcasted_iota(jnp.int32, (NN, NN), 1)
    e0 = jnp.dot((uu == col).astype(jnp.float32), lapn,
                 preferred_element_type=jnp.float32)
    e1 = jnp.dot((vv == col).astype(jnp.float32), lapn,
                 preferred_element_type=jnp.float32)
    u = jnp.concatenate([lapn, e0], axis=0)
    v = jnp.concatenate([lapn, e1], axis=0)
    eqf = jnp.concatenate(
        [jnp.ones((NN, 1), jnp.float32),
         (uu == vv).astype(jnp.float32)], axis=0)
    wm = w_ref[...]
    dn = (((1,), (1,)), ((), ()))
    le = (lax.dot_general(u, wm[:, :K], dn,
                          preferred_element_type=jnp.float32)
          + lax.dot_general(v, wm[:, K:], dn,
                            preferred_element_type=jnp.float32))
    o0 = ord_ref[0:1, :]
    oe = o0 + eqf * (ord_ref[1:2, :] - o0)
    out_ref[0] = feat_ref[...] + le + oe


def _sc_gather(half, nd, ed, atom_emb, edge_emb):
    mesh = plsc.VectorSubcoreMesh(core_axis_name="c", subcore_axis_name="s")
    return pl.kernel(
        _make_sc_body(half),
        out_type=jax.ShapeDtypeStruct((HB * ACT, D), jnp.float32),
        mesh=mesh,
        scratch_types=[
            pltpu.VMEM((TPW,), jnp.int32),
            pltpu.VMEM((TPW,), jnp.int32),
            pltpu.VMEM((C, D), jnp.float32),
            pltpu.VMEM((C, D), jnp.float32),
            pltpu.SemaphoreType.DMA,
            pltpu.SemaphoreType.DMA,
            pltpu.SemaphoreType.DMA,
            pltpu.SemaphoreType.DMA,
        ],
    )(nd, ed, atom_emb, edge_emb)


def _tc_combine(half, lapf, euv, feat, lapw, ordm, prev):
    off = half * HB
    return pl.pallas_call(
        _tc_body,
        grid=(HB,),
        in_specs=[
            pl.BlockSpec((NN, K), lambda b: (b + off, 0)),
            pl.BlockSpec((NN, 16), lambda b: (b + off, 0)),
            pl.BlockSpec((ACT, D), lambda b: (b, 0)),
            pl.BlockSpec((D, 2 * K), lambda b: (0, 0)),
            pl.BlockSpec((2, D), lambda b: (0, 0)),
            pl.BlockSpec(memory_space=pl.ANY),
        ],
        out_specs=pl.BlockSpec((1, ACT, D), lambda b: (b + off, 0, 0)),
        out_shape=jax.ShapeDtypeStruct((B, MAXLEN, D), jnp.float32),
        input_output_aliases={5: 0},
    )(lapf, euv, feat, lapw, ordm, prev)


def kernel(node_data, node_num, lap_eigvec, edge_index, edge_data, edge_num,
           atom_emb, edge_emb, lap_W, order_emb):
    nd = node_data.reshape(B * NN).astype(jnp.int32)
    ed = edge_data.reshape(B * EN).astype(jnp.int32)

    ei = edge_index.astype(jnp.int32)
    euv = jnp.repeat(
        ei.reshape(2, B, EN)[:, :, :NN].transpose(1, 2, 0).reshape(NR, 2),
        8, axis=1)

    lapf = lap_eigvec.astype(jnp.float32)
    lapw = lap_W.astype(jnp.float32)
    ordm = order_emb.astype(jnp.float32)

    feat_a = _sc_gather(0, nd, ed, atom_emb, edge_emb)
    feat_b = _sc_gather(1, nd, ed, atom_emb, edge_emb)

    zhalf = pl.pallas_call(
        _tc_zero_body,
        grid=(B,),
        out_specs=pl.BlockSpec((1, ACT, D), lambda b: (b, 1, 0)),
        out_shape=jax.ShapeDtypeStruct((B, MAXLEN, D), jnp.float32),
    )()

    out = _tc_combine(0, lapf, euv, feat_a, lapw, ordm, zhalf)
    out = _tc_combine(1, lapf, euv, feat_b, lapw, ordm, out)
    return out
